# baseline (speedup 1.0000x reference)
import jax
import jax.numpy as jnp
from jax import lax
from jax.experimental import pallas as pl
from jax.experimental.pallas import tpu as pltpu

NUM_CLASSES = 80
_LN2 = 0.6931471805599453


def _body(tgt_ref, x_ref, lp_ref, lt_ref, out_ref):
    x = x_ref[0]
    tgt = tgt_ref[0]
    lp = lp_ref[0]
    lt = lt_ref[0]

    cls_id = lax.broadcasted_iota(jnp.int32, (NUM_CLASSES, 1), 0) + 1
    t = tgt == cls_id

    s1 = jnp.where(t, 1.0, -1.0)
    u = x * s1 + 0.5
    q = jnp.maximum(1.0 + jnp.tanh(u), 1e-38)
    lg = jnp.log2(q)
    wc = s1 * -0.125 + 0.25
    cls_neg_part = jnp.sum(wc * lg)

    pos = tgt > 0
    np_part = jnp.sum(jnp.where(pos, 1.0, 0.0))

    d = lp - lt
    ad = jnp.abs(d)
    elem = jnp.where(ad < 1.0, 0.5 * d * d, ad - 0.5)
    loc_part = jnp.sum(jnp.where(pos, elem, 0.0))

    @pl.when((pl.program_id(0) == 0) & (pl.program_id(1) == 0))
    def _():
        out_ref[0] = 0.0
        out_ref[1] = 0.0
        out_ref[2] = 0.0

    out_ref[0] += cls_neg_part
    out_ref[1] += loc_part
    out_ref[2] += np_part


def kernel(loc_preds, loc_targets, cls_preds, cls_targets):
    b, a, _ = loc_preds.shape

    xt = cls_preds.transpose(0, 2, 1)
    lpt = loc_preds.transpose(0, 2, 1)
    ltt = loc_targets.transpose(0, 2, 1)
    tgt3 = cls_targets.reshape(b, 1, a)

    lblk = 8192
    grid = (b, a // lblk)

    sums = pl.pallas_call(
        _body,
        grid=grid,
        in_specs=[
            pl.BlockSpec((1, 1, lblk), lambda i, j: (i, 0, j)),
            pl.BlockSpec((1, NUM_CLASSES, lblk), lambda i, j: (i, 0, j)),
            pl.BlockSpec((1, 4, lblk), lambda i, j: (i, 0, j)),
            pl.BlockSpec((1, 4, lblk), lambda i, j: (i, 0, j)),
        ],
        out_specs=pl.BlockSpec(memory_space=pltpu.SMEM),
        out_shape=jax.ShapeDtypeStruct((3,), jnp.float32),
    )(tgt3, xt, lpt, ltt)

    num_pos = sums[2]
    n_el = b * a * NUM_CLASSES
    cls_loss = _LN2 * (0.375 * n_el - 0.25 * num_pos - sums[0])
    return (cls_loss + sums[1]) / num_pos


# speedup vs baseline: 1.0620x; 1.0620x over previous
import jax
import jax.numpy as jnp
from jax import lax
from jax.experimental import pallas as pl
from jax.experimental.pallas import tpu as pltpu

NUM_CLASSES = 80
_LN2 = 0.6931471805599453


def _body(tgt_ref, x_ref, lp_ref, lt_ref, out_ref):
    x = x_ref[0]
    tgt = tgt_ref[0]
    lp = lp_ref[0]
    lt = lt_ref[0]

    q0 = jnp.maximum(1.0 + jnp.tanh(0.5 - x), 1e-38)
    lg0_sum = jnp.sum(jnp.log(q0))

    cls_id = lax.broadcasted_iota(jnp.int32, (NUM_CLASSES, 1), 0) + 1
    t = tgt == cls_id
    x_star = jnp.sum(jnp.where(t, x, 0.0), axis=0, keepdims=True)

    pos = tgt > 0
    posf = jnp.where(pos, 1.0, 0.0)
    q1 = jnp.maximum(1.0 + jnp.tanh(x_star + 0.5), 1e-38)
    q0s = jnp.maximum(1.0 + jnp.tanh(0.5 - x_star), 1e-38)
    corr = jnp.sum(posf * (0.125 * jnp.log(q1) - 0.375 * jnp.log(q0s)))
    cls_neg_part = 0.375 * lg0_sum + corr

    np_part = jnp.sum(posf)

    d = lp - lt
    ad = jnp.abs(d)
    elem = jnp.where(ad < 1.0, 0.5 * d * d, ad - 0.5)
    loc_part = jnp.sum(jnp.where(pos, elem, 0.0))

    @pl.when((pl.program_id(0) == 0) & (pl.program_id(1) == 0))
    def _():
        out_ref[0] = 0.0
        out_ref[1] = 0.0
        out_ref[2] = 0.0

    out_ref[0] += cls_neg_part
    out_ref[1] += loc_part
    out_ref[2] += np_part


def kernel(loc_preds, loc_targets, cls_preds, cls_targets):
    b, a, _ = loc_preds.shape

    xt = cls_preds.transpose(0, 2, 1)
    lpt = loc_preds.transpose(0, 2, 1)
    ltt = loc_targets.transpose(0, 2, 1)
    tgt3 = cls_targets.reshape(b, 1, a)

    lblk = 8192
    grid = (b, a // lblk)

    sums = pl.pallas_call(
        _body,
        grid=grid,
        in_specs=[
            pl.BlockSpec((1, 1, lblk), lambda i, j: (i, 0, j)),
            pl.BlockSpec((1, NUM_CLASSES, lblk), lambda i, j: (i, 0, j)),
            pl.BlockSpec((1, 4, lblk), lambda i, j: (i, 0, j)),
            pl.BlockSpec((1, 4, lblk), lambda i, j: (i, 0, j)),
        ],
        out_specs=pl.BlockSpec(memory_space=pltpu.SMEM),
        out_shape=jax.ShapeDtypeStruct((3,), jnp.float32),
    )(tgt3, xt, lpt, ltt)

    num_pos = sums[2]
    n_el = b * a * NUM_CLASSES
    cls_loss = _LN2 * (0.375 * n_el - 0.25 * num_pos) - sums[0]
    return (cls_loss + sums[1]) / num_pos


# speedup vs baseline: 1.1126x; 1.0477x over previous
import jax
import jax.numpy as jnp
from jax import lax
from jax.experimental import pallas as pl
from jax.experimental.pallas import tpu as pltpu

NUM_CLASSES = 80
_LN2 = 0.6931471805599453


def _body(tgt_ref, x_ref, lp_ref, lt_ref, out_ref):
    x = x_ref[0]
    tgt = tgt_ref[0]
    lp = lp_ref[0]
    lt = lt_ref[0]

    q0 = jnp.maximum(1.0 + jnp.tanh(0.5 - x), 1e-38)
    lg0_sum = jnp.sum(jnp.log(q0))

    cls_id = lax.broadcasted_iota(jnp.int32, (NUM_CLASSES, 1), 0) + 1
    t = tgt == cls_id
    x_star = jnp.sum(jnp.where(t, x, 0.0), axis=0, keepdims=True)

    pos = tgt > 0
    posf = jnp.where(pos, 1.0, 0.0)
    q1 = jnp.maximum(1.0 + jnp.tanh(x_star + 0.5), 1e-38)
    q0s = jnp.maximum(1.0 + jnp.tanh(0.5 - x_star), 1e-38)
    corr = jnp.sum(posf * (0.125 * jnp.log(q1) - 0.375 * jnp.log(q0s)))
    cls_neg_part = 0.375 * lg0_sum + corr

    np_part = jnp.sum(posf)

    d = lp - lt
    ad = jnp.abs(d)
    elem = jnp.where(ad < 1.0, 0.5 * d * d, ad - 0.5)
    loc_part = jnp.sum(jnp.where(pos, elem, 0.0))

    @pl.when((pl.program_id(0) == 0) & (pl.program_id(1) == 0))
    def _():
        out_ref[0] = 0.0
        out_ref[1] = 0.0
        out_ref[2] = 0.0

    out_ref[0] += cls_neg_part
    out_ref[1] += loc_part
    out_ref[2] += np_part


def kernel(loc_preds, loc_targets, cls_preds, cls_targets):
    b, a, _ = loc_preds.shape

    xt = cls_preds.transpose(0, 2, 1)
    lpt = loc_preds.transpose(0, 2, 1)
    ltt = loc_targets.transpose(0, 2, 1)
    tgt3 = cls_targets.reshape(b, 1, a)

    lblk = 16384
    grid = (b, a // lblk)

    sums = pl.pallas_call(
        _body,
        grid=grid,
        in_specs=[
            pl.BlockSpec((1, 1, lblk), lambda i, j: (i, 0, j)),
            pl.BlockSpec((1, NUM_CLASSES, lblk), lambda i, j: (i, 0, j)),
            pl.BlockSpec((1, 4, lblk), lambda i, j: (i, 0, j)),
            pl.BlockSpec((1, 4, lblk), lambda i, j: (i, 0, j)),
        ],
        out_specs=pl.BlockSpec(memory_space=pltpu.SMEM),
        out_shape=jax.ShapeDtypeStruct((3,), jnp.float32),
    )(tgt3, xt, lpt, ltt)

    num_pos = sums[2]
    n_el = b * a * NUM_CLASSES
    cls_loss = _LN2 * (0.375 * n_el - 0.25 * num_pos) - sums[0]
    return (cls_loss + sums[1]) / num_pos


# speedup vs baseline: 1.1295x; 1.0151x over previous
import jax
import jax.numpy as jnp
from jax import lax
from jax.experimental import pallas as pl
from jax.experimental.pallas import tpu as pltpu

NUM_CLASSES = 80
_LN2 = 0.6931471805599453


def _body(tgt_ref, x_ref, lp_ref, lt_ref, out_ref):
    x = x_ref[0]
    tgt = tgt_ref[0]
    lp = lp_ref[0]
    lt = lt_ref[0]

    q0 = jnp.maximum(1.0 + jnp.tanh(0.5 - x), 1e-38)
    lg0_sum = jnp.sum(jnp.log(q0))

    cls_id = lax.broadcasted_iota(jnp.int32, (NUM_CLASSES, 1), 0) + 1
    t = tgt == cls_id
    x_star = jnp.sum(jnp.where(t, x, 0.0), axis=0, keepdims=True)

    pos = tgt > 0
    posf = jnp.where(pos, 1.0, 0.0)
    q1 = jnp.maximum(1.0 + jnp.tanh(x_star + 0.5), 1e-38)
    q0s = jnp.maximum(1.0 + jnp.tanh(0.5 - x_star), 1e-38)
    corr = jnp.sum(posf * (0.125 * jnp.log(q1) - 0.375 * jnp.log(q0s)))
    cls_neg_part = 0.375 * lg0_sum + corr

    np_part = jnp.sum(posf)

    d = lp - lt
    ad = jnp.abs(d)
    elem = jnp.where(ad < 1.0, 0.5 * d * d, ad - 0.5)
    loc_part = jnp.sum(jnp.where(pos, elem, 0.0))

    @pl.when((pl.program_id(0) == 0) & (pl.program_id(1) == 0))
    def _():
        out_ref[0] = 0.0
        out_ref[1] = 0.0
        out_ref[2] = 0.0

    out_ref[0] += cls_neg_part
    out_ref[1] += loc_part
    out_ref[2] += np_part


def kernel(loc_preds, loc_targets, cls_preds, cls_targets):
    b, a, _ = loc_preds.shape

    xt = cls_preds.transpose(0, 2, 1)
    lpt = loc_preds.transpose(0, 2, 1)
    ltt = loc_targets.transpose(0, 2, 1)
    tgt3 = cls_targets.reshape(b, 1, a)

    lblk = 32768
    grid = (b, a // lblk)

    sums = pl.pallas_call(
        _body,
        grid=grid,
        in_specs=[
            pl.BlockSpec((1, 1, lblk), lambda i, j: (i, 0, j)),
            pl.BlockSpec((1, NUM_CLASSES, lblk), lambda i, j: (i, 0, j)),
            pl.BlockSpec((1, 4, lblk), lambda i, j: (i, 0, j)),
            pl.BlockSpec((1, 4, lblk), lambda i, j: (i, 0, j)),
        ],
        out_specs=pl.BlockSpec(memory_space=pltpu.SMEM),
        out_shape=jax.ShapeDtypeStruct((3,), jnp.float32),
    )(tgt3, xt, lpt, ltt)

    num_pos = sums[2]
    n_el = b * a * NUM_CLASSES
    cls_loss = _LN2 * (0.375 * n_el - 0.25 * num_pos) - sums[0]
    return (cls_loss + sums[1]) / num_pos


# speedup vs baseline: 1.1373x; 1.0069x over previous
import jax
import jax.numpy as jnp
from jax import lax
from jax.experimental import pallas as pl
from jax.experimental.pallas import tpu as pltpu

NUM_CLASSES = 80
BB = 2
_LN2 = 0.6931471805599453


def _body(tgt_ref, x_ref, lp_ref, lt_ref, out_ref):
    cls_neg_part = 0.0
    loc_part = 0.0
    np_part = 0.0
    cls_id = lax.broadcasted_iota(jnp.int32, (NUM_CLASSES, 1), 0) + 1

    for k in range(BB):
        x = x_ref[k]
        tgt = tgt_ref[k]
        lp = lp_ref[k]
        lt = lt_ref[k]

        q0 = jnp.maximum(1.0 + jnp.tanh(0.5 - x), 1e-38)
        lg0_sum = jnp.sum(jnp.log(q0))

        t = tgt == cls_id
        x_star = jnp.sum(jnp.where(t, x, 0.0), axis=0, keepdims=True)

        pos = tgt > 0
        posf = jnp.where(pos, 1.0, 0.0)
        q1 = jnp.maximum(1.0 + jnp.tanh(x_star + 0.5), 1e-38)
        q0s = jnp.maximum(1.0 + jnp.tanh(0.5 - x_star), 1e-38)
        corr = jnp.sum(posf * (0.125 * jnp.log(q1)
                               - 0.375 * jnp.log(q0s)))
        cls_neg_part += 0.375 * lg0_sum + corr

        np_part += jnp.sum(posf)

        d = lp - lt
        ad = jnp.abs(d)
        elem = jnp.where(ad < 1.0, 0.5 * d * d, ad - 0.5)
        loc_part += jnp.sum(jnp.where(pos, elem, 0.0))

    @pl.when(pl.program_id(0) == 0)
    def _():
        out_ref[0] = 0.0
        out_ref[1] = 0.0
        out_ref[2] = 0.0

    out_ref[0] += cls_neg_part
    out_ref[1] += loc_part
    out_ref[2] += np_part


def kernel(loc_preds, loc_targets, cls_preds, cls_targets):
    b, a, _ = loc_preds.shape

    xt = cls_preds.transpose(0, 2, 1)
    lpt = loc_preds.transpose(0, 2, 1)
    ltt = loc_targets.transpose(0, 2, 1)
    tgt3 = cls_targets.reshape(b, 1, a)

    grid = (b // BB,)

    sums = pl.pallas_call(
        _body,
        grid=grid,
        in_specs=[
            pl.BlockSpec((BB, 1, a), lambda i: (i, 0, 0)),
            pl.BlockSpec((BB, NUM_CLASSES, a), lambda i: (i, 0, 0)),
            pl.BlockSpec((BB, 4, a), lambda i: (i, 0, 0)),
            pl.BlockSpec((BB, 4, a), lambda i: (i, 0, 0)),
        ],
        out_specs=pl.BlockSpec(memory_space=pltpu.SMEM),
        out_shape=jax.ShapeDtypeStruct((3,), jnp.float32),
    )(tgt3, xt, lpt, ltt)

    num_pos = sums[2]
    n_el = b * a * NUM_CLASSES
    cls_loss = _LN2 * (0.375 * n_el - 0.25 * num_pos) - sums[0]
    return (cls_loss + sums[1]) / num_pos
